# baseline (speedup 1.0000x reference)
import functools

import jax
import jax.numpy as jnp
from jax import lax
from jax.experimental import pallas as pl
from jax.experimental.pallas import tpu as pltpu
from jax.experimental.pallas import tpu_sc as plsc

BATCH = 16384
DIM = 64
LANES = 16


@functools.cache
def _build():
    info = plsc.get_sparse_core_info()
    nc, ns = info.num_cores, info.num_subcores
    nw = nc * ns
    b_per_w = BATCH // nw
    n_grp = b_per_w // LANES
    mesh = plsc.VectorSubcoreMesh(core_axis_name="c", subcore_axis_name="s")

    @functools.partial(
        pl.kernel,
        mesh=mesh,
        out_type=jax.ShapeDtypeStruct((BATCH, DIM), jnp.float32),
        scratch_types=[
            pltpu.VMEM((b_per_w,), jnp.int32),
            pltpu.VMEM((b_per_w, DIM), jnp.float32),
            pltpu.SemaphoreType.DMA,
        ],
        compiler_params=pltpu.CompilerParams(skip_device_barrier=True),
    )
    def gather_kernel(idx_hbm, table_hbm, out_hbm, idx_v, rows_v, sem):
        wid = lax.axis_index("s") * nc + lax.axis_index("c")
        base = wid * b_per_w
        pltpu.sync_copy(idx_hbm.at[pl.ds(base, b_per_w)], idx_v)

        def group(g, _):
            ids = idx_v[pl.ds(g * LANES, LANES)]
            copies = []
            for j in range(LANES):
                row = ids[j]
                copies.append(
                    pltpu.async_copy(
                        table_hbm.at[pl.ds(row, 1)],
                        rows_v.at[pl.ds(g * LANES + j, 1)],
                        sem,
                    )
                )
            for cp in copies:
                cp.wait()
            return _

        lax.fori_loop(0, n_grp, group, 0)
        pltpu.sync_copy(rows_v, out_hbm.at[pl.ds(base, b_per_w)])

    return gather_kernel


def kernel(problem_id, embedding_table):
    gather_kernel = _build()
    return gather_kernel(problem_id, embedding_table)


# speedup vs baseline: 1.9513x; 1.9513x over previous
import functools

import jax
import jax.numpy as jnp
from jax import lax
from jax.experimental import pallas as pl
from jax.experimental.pallas import tpu as pltpu
from jax.experimental.pallas import tpu_sc as plsc

BATCH = 16384
DIM = 64
BLK = 128
RING = 8


@functools.cache
def _build():
    info = plsc.get_sparse_core_info()
    nc, ns = info.num_cores, info.num_subcores
    nw = nc * ns
    b_per_w = BATCH // nw
    mesh = plsc.VectorSubcoreMesh(core_axis_name="c", subcore_axis_name="s")

    @functools.partial(
        pl.kernel,
        mesh=mesh,
        out_type=jax.ShapeDtypeStruct((DIM, BATCH), jnp.float32),
        scratch_types=[
            pltpu.VMEM((b_per_w,), jnp.int32),
            pltpu.VMEM((b_per_w + 16,), jnp.int32),
            pltpu.VMEM((b_per_w + 16,), jnp.int32),
            pltpu.VMEM((RING, DIM, BLK), jnp.float32),
            pltpu.VMEM((DIM, BLK), jnp.float32),
            pltpu.SemaphoreType.DMA,
        ],
        compiler_params=pltpu.CompilerParams(needs_layout_passes=False),
    )
    def gather_kernel(idx_hbm, tab_t_hbm, out_t_hbm, idx_v, lane0_v, rr_v,
                      blk_v, buf_v, sem):
        wid = lax.axis_index("s") * nc + lax.axis_index("c")
        base = wid * b_per_w
        pltpu.sync_copy(idx_hbm.at[pl.ds(base, b_per_w)], idx_v)
        lanes = lax.iota(jnp.int32, 16)

        def prep(g, _):
            ids = idx_v[pl.ds(g * 16, 16)]
            lane0 = (ids >> 7) << 7
            lane0_v[pl.ds(g * 16, 16)] = lane0
            rr_v[pl.ds(g * 16, 16)] = ids - lane0
            return _

        lax.fori_loop(0, b_per_w // 16, prep, 0)

        def fire(s):
            block0 = pl.multiple_of(lane0_v[pl.ds(s, 16)][0], BLK)
            slot = lax.rem(s, RING)
            pltpu.async_copy(
                tab_t_hbm.at[:, pl.ds(block0, BLK)], blk_v.at[slot], sem
            )

        def select(s):
            slot = lax.rem(s, RING)
            pltpu.make_async_copy(
                tab_t_hbm.at[:, pl.ds(0, BLK)], blk_v.at[slot], sem
            ).wait()
            r16 = jnp.zeros((16,), jnp.int32) + rr_v[pl.ds(s, 16)][0]
            s16 = jnp.zeros((16,), jnp.int32) + slot
            col = jnp.zeros((16,), jnp.int32) + lax.rem(s, BLK)
            for k in range(DIM // 16):
                c16 = lanes + (k * 16)
                val = plsc.load_gather(blk_v, [s16, c16, r16])
                plsc.store_scatter(buf_v, [c16, col], val)

            @pl.when(lax.rem(s, BLK) == BLK - 1)
            def _flush():
                q0 = pl.multiple_of((s >> 7) << 7, BLK)
                pltpu.sync_copy(
                    buf_v, out_t_hbm.at[:, pl.ds(base + q0, BLK)]
                )

        def prologue(s, _):
            fire(s)
            return _

        lax.fori_loop(0, RING, prologue, 0)

        def steady(s, _):
            fire(s)
            select(s - RING)
            return _

        lax.fori_loop(RING, b_per_w, steady, 0)

        def epilogue(s, _):
            select(s)
            return _

        lax.fori_loop(b_per_w - RING, b_per_w, epilogue, 0)

    return gather_kernel


def kernel(problem_id, embedding_table):
    gather_kernel = _build()
    out_t = gather_kernel(problem_id, embedding_table.T)
    return out_t.T


# speedup vs baseline: 1.9831x; 1.0163x over previous
import functools

import jax
import jax.numpy as jnp
from jax import lax
from jax.experimental import pallas as pl
from jax.experimental.pallas import tpu as pltpu
from jax.experimental.pallas import tpu_sc as plsc

BATCH = 16384
DIM = 64
BLK = 128
RING = 8


@functools.cache
def _build():
    info = plsc.get_sparse_core_info()
    nc, ns = info.num_cores, info.num_subcores
    nw = nc * ns
    b_per_w = BATCH // nw
    mesh = plsc.VectorSubcoreMesh(core_axis_name="c", subcore_axis_name="s")

    @functools.partial(
        pl.kernel,
        mesh=mesh,
        out_type=jax.ShapeDtypeStruct((DIM, BATCH), jnp.float32),
        scratch_types=[
            pltpu.VMEM((b_per_w,), jnp.int32),
            pltpu.VMEM((b_per_w + 16,), jnp.int32),
            pltpu.VMEM((b_per_w + 16,), jnp.int32),
            pltpu.VMEM((RING, DIM, BLK), jnp.float32),
            pltpu.VMEM((DIM, BLK), jnp.float32),
            pltpu.SemaphoreType.DMA,
        ],
        compiler_params=pltpu.CompilerParams(needs_layout_passes=False),
    )
    def gather_kernel(idx_hbm, tab_t_hbm, out_t_hbm, idx_v, lane0_v, rr_v,
                      blk_v, buf_v, sem):
        wid = lax.axis_index("s") * nc + lax.axis_index("c")
        base = wid * b_per_w
        pltpu.sync_copy(idx_hbm.at[pl.ds(base, b_per_w)], idx_v)
        lanes = lax.iota(jnp.int32, 16)

        def prep(g, _):
            ids = idx_v[pl.ds(g * 16, 16)]
            lane0 = (ids >> 7) << 7
            lane0_v[pl.ds(g * 16, 16)] = lane0
            rr_v[pl.ds(g * 16, 16)] = ids - lane0
            return _

        lax.fori_loop(0, b_per_w // 16, prep, 0)

        def fire(s):
            block0 = pl.multiple_of(lane0_v[pl.ds(s, 16)][0], BLK)
            slot = lax.rem(s, RING)
            pltpu.async_copy(
                tab_t_hbm.at[:, pl.ds(block0, BLK)], blk_v.at[slot], sem
            )

        def select(s):
            slot = lax.rem(s, RING)
            pltpu.make_async_copy(
                tab_t_hbm.at[:, pl.ds(0, BLK)], blk_v.at[slot], sem
            ).wait()
            r16 = jnp.zeros((16,), jnp.int32) + rr_v[pl.ds(s, 16)][0]
            s16 = jnp.zeros((16,), jnp.int32) + slot
            col = jnp.zeros((16,), jnp.int32) + lax.rem(s, BLK)
            for k in range(DIM // 16):
                c16 = lanes + (k * 16)
                val = plsc.load_gather(blk_v, [s16, c16, r16])
                plsc.store_scatter(buf_v, [c16, col], val)

            @pl.when(lax.rem(s, BLK) == BLK - 1)
            def _flush():
                q0 = pl.multiple_of((s >> 7) << 7, BLK)
                pltpu.sync_copy(
                    buf_v, out_t_hbm.at[:, pl.ds(base + q0, BLK)]
                )

        def prologue(s, _):
            fire(s)
            return _

        lax.fori_loop(0, RING, prologue, 0)

        def steady(s, _):
            select(s - RING)
            fire(s)
            return _

        lax.fori_loop(RING, b_per_w, steady, 0)

        def epilogue(s, _):
            select(s)
            return _

        lax.fori_loop(b_per_w - RING, b_per_w, epilogue, 0)

    return gather_kernel


def kernel(problem_id, embedding_table):
    gather_kernel = _build()
    out_t = gather_kernel(problem_id, embedding_table.T)
    return out_t.T


# speedup vs baseline: 2.4817x; 1.2514x over previous
import functools

import jax
import jax.numpy as jnp
from jax import lax
from jax.experimental import pallas as pl
from jax.experimental.pallas import tpu as pltpu
from jax.experimental.pallas import tpu_sc as plsc

BATCH = 16384
DIM = 64
NBLK = 7813
RING = 4
OUT_RING = 16


@functools.cache
def _build():
    info = plsc.get_sparse_core_info()
    nc, ns = info.num_cores, info.num_subcores
    nw = nc * ns
    nb_base = NBLK // nw
    extra = NBLK % nw
    mesh = plsc.VectorSubcoreMesh(core_axis_name="c", subcore_axis_name="s")

    @functools.partial(
        pl.kernel,
        mesh=mesh,
        out_type=jax.ShapeDtypeStruct((BATCH, DIM), jnp.float32),
        scratch_types=[
            pltpu.VMEM((BATCH,), jnp.int32),
            pltpu.VMEM((BATCH + 16,), jnp.int32),
            pltpu.VMEM((BATCH + 16,), jnp.int32),
            pltpu.VMEM((BATCH + 16,), jnp.int32),
            pltpu.VMEM((BATCH + 16,), jnp.int32),
            pltpu.VMEM((272,), jnp.int32),
            pltpu.VMEM((272,), jnp.int32),
            pltpu.VMEM((272,), jnp.int32),
            pltpu.VMEM((RING, DIM, 128), jnp.float32),
            pltpu.VMEM((OUT_RING, DIM), jnp.float32),
            pltpu.SemaphoreType.DMA,
            pltpu.SemaphoreType.DMA,
        ],
        compiler_params=pltpu.CompilerParams(needs_layout_passes=False),
    )
    def gather_kernel(idx_hbm, tab_t_hbm, out_hbm, idx_full, ent_id, ent_pos,
                      srt_id, srt_pos, counts, offs, fill, chunk_v, row_v,
                      sem, sem_out):
        wid = lax.axis_index("s") * nc + lax.axis_index("c")
        lo_b = wid * nb_base + jnp.minimum(wid, extra)
        n_b = nb_base + (wid < extra).astype(jnp.int32)
        pltpu.sync_copy(idx_hbm, idx_full)
        lanes = lax.iota(jnp.int32, 16)
        zeros = jnp.zeros((16,), jnp.int32)
        lane0_mask = lanes == 0

        def scan_body(g, n):
            ids = idx_full[pl.ds(g * 16, 16)]
            blk = ids >> 7
            m = (blk >= lo_b) & (blk < lo_b + n_b)
            plsc.store_compressed(ent_id.at[pl.ds(n, 16)], ids, mask=m)
            plsc.store_compressed(
                ent_pos.at[pl.ds(n, 16)], g * 16 + lanes, mask=m
            )
            return n + plsc.all_reduce_population_count(m)[0]

        n_ent = lax.fori_loop(0, BATCH // 16, scan_body, 0)

        for k in range(17):
            counts[pl.ds(k * 16, 16)] = zeros

        def count_body(e, _):
            b16 = zeros + ((ent_id[pl.ds(e, 16)][0] >> 7) - lo_b)
            c = plsc.load_gather(counts, [b16])
            plsc.store_scatter(counts, [b16], c + 1, mask=lane0_mask)
            return _

        lax.fori_loop(0, n_ent, count_body, 0)

        carry = jnp.int32(0)
        for k in range(16):
            cv = counts[pl.ds(k * 16, 16)]
            s = plsc.cumsum(cv)
            start = s - cv + carry
            offs[pl.ds(k * 16, 16)] = start
            fill[pl.ds(k * 16, 16)] = start
            carry = carry + s[15]

        def place_body(e, _):
            idv = ent_id[pl.ds(e, 16)][0]
            pv = ent_pos[pl.ds(e, 16)][0]
            b16 = zeros + ((idv >> 7) - lo_b)
            o = plsc.load_gather(fill, [b16])
            plsc.store_scatter(srt_id, [o], zeros + idv, mask=lane0_mask)
            plsc.store_scatter(srt_pos, [o], zeros + pv, mask=lane0_mask)
            plsc.store_scatter(fill, [b16], o + 1, mask=lane0_mask)
            return _

        lax.fori_loop(0, n_ent, place_body, 0)

        def fire_blk(bl):
            lane0 = pl.multiple_of((lo_b + bl) << 7, 128)
            pltpu.async_copy(
                tab_t_hbm.at[:, pl.ds(lane0, 128)],
                chunk_v.at[bl & (RING - 1)],
                sem,
            )

        def prologue(bl, _):
            fire_blk(bl)
            return _

        lax.fori_loop(0, RING, prologue, 0)

        def blk_body(bl, processed):
            slot = bl & (RING - 1)
            pltpu.make_async_copy(
                tab_t_hbm.at[:, pl.ds(0, 128)], chunk_v.at[slot], sem
            ).wait()
            st = offs[pl.ds(bl, 16)][0]
            ne = counts[pl.ds(bl, 16)][0]
            s16 = zeros + slot

            def e_body(e, pr):
                idv = srt_id[pl.ds(e, 16)][0]
                pv = srt_pos[pl.ds(e, 16)][0]
                r16 = zeros + (idv & 127)
                rslot = pr & (OUT_RING - 1)

                @pl.when(pr >= OUT_RING)
                def _drain_one():
                    pltpu.make_async_copy(
                        out_hbm.at[0], row_v.at[rslot], sem_out
                    ).wait()

                for k in range(DIM // 16):
                    c16 = lanes + (k * 16)
                    val = plsc.load_gather(chunk_v, [s16, c16, r16])
                    row_v[rslot, pl.ds(k * 16, 16)] = val
                pltpu.async_copy(row_v.at[rslot], out_hbm.at[pv], sem_out)
                return pr + 1

            processed = lax.fori_loop(st, st + ne, e_body, processed)

            @pl.when(bl + RING < n_b)
            def _refill():
                fire_blk(bl + RING)

            return processed

        processed = lax.fori_loop(0, n_b, blk_body, 0)

        def drain_body(k, _):
            @pl.when(k < jnp.minimum(processed, OUT_RING))
            def _d():
                pltpu.make_async_copy(
                    out_hbm.at[0], row_v.at[k & (OUT_RING - 1)], sem_out
                ).wait()

            return _

        lax.fori_loop(0, OUT_RING, drain_body, 0)

    return gather_kernel


def kernel(problem_id, embedding_table):
    gather_kernel = _build()
    return gather_kernel(problem_id, embedding_table.T)


# speedup vs baseline: 2.5637x; 1.0330x over previous
import functools

import jax
import jax.numpy as jnp
from jax import lax
from jax.experimental import pallas as pl
from jax.experimental.pallas import tpu as pltpu
from jax.experimental.pallas import tpu_sc as plsc

BATCH = 16384
DIM = 64
NBLK = 7813
NB = 244
CHW = 512
RING = 2
OUT_RING = 16


@functools.cache
def _build():
    info = plsc.get_sparse_core_info()
    nc, ns = info.num_cores, info.num_subcores
    nw = nc * ns
    mesh = plsc.VectorSubcoreMesh(core_axis_name="c", subcore_axis_name="s")

    @functools.partial(
        pl.kernel,
        mesh=mesh,
        out_type=jax.ShapeDtypeStruct((BATCH, DIM), jnp.float32),
        scratch_types=[
            pltpu.VMEM((BATCH,), jnp.int32),
            pltpu.VMEM((BATCH + 16,), jnp.int32),
            pltpu.VMEM((BATCH + 16,), jnp.int32),
            pltpu.VMEM((272,), jnp.int32),
            pltpu.VMEM((272,), jnp.int32),
            pltpu.VMEM((272,), jnp.int32),
            pltpu.VMEM((RING, DIM, CHW), jnp.float32),
            pltpu.VMEM((OUT_RING, DIM), jnp.float32),
            pltpu.SemaphoreType.DMA,
            pltpu.SemaphoreType.DMA,
        ],
        compiler_params=pltpu.CompilerParams(needs_layout_passes=False),
    )
    def gather_kernel(idx_hbm, tab_t_hbm, out_hbm, idx_full, ent_w, srt_w,
                      counts, offs, fill, chunk_v, row_v, sem, sem_out):
        wid = lax.axis_index("s") * nc + lax.axis_index("c")
        last = wid == nw - 1
        lo_b = wid * NB
        n_b = NB + 5 * last.astype(jnp.int32)
        pltpu.sync_copy(idx_hbm, idx_full)
        lanes = lax.iota(jnp.int32, 16)
        zeros = jnp.zeros((16,), jnp.int32)
        lane0_mask = lanes == 0

        def scan_body(g, n):
            ids = idx_full[pl.ds(g * 16, 16)]
            blk = ids >> 7
            m = (blk >= lo_b) & (blk < lo_b + n_b)
            packed = (
                ((blk - lo_b) << 21) | ((ids & 127) << 14) | (g * 16 + lanes)
            )
            plsc.store_compressed(ent_w.at[pl.ds(n, 16)], packed, mask=m)
            return n + plsc.all_reduce_population_count(m)[0]

        n_ent = lax.fori_loop(0, BATCH // 16, scan_body, 0)

        for k in range(17):
            counts[pl.ds(k * 16, 16)] = zeros

        def count_body(e, _):
            b16 = zeros + (ent_w[pl.ds(e, 16)][0] >> 21)
            c = plsc.load_gather(counts, [b16])
            plsc.store_scatter(counts, [b16], c + 1, mask=lane0_mask)
            return _

        lax.fori_loop(0, n_ent, count_body, 0)

        carry = jnp.int32(0)
        for k in range(16):
            cv = counts[pl.ds(k * 16, 16)]
            s = plsc.cumsum(cv)
            start = s - cv + carry
            offs[pl.ds(k * 16, 16)] = start
            fill[pl.ds(k * 16, 16)] = start
            carry = carry + s[15]

        def place_body(e, _):
            w = ent_w[pl.ds(e, 16)][0]
            b16 = zeros + (w >> 21)
            o = plsc.load_gather(fill, [b16])
            plsc.store_scatter(srt_w, [o], zeros + w, mask=lane0_mask)
            plsc.store_scatter(fill, [b16], o + 1, mask=lane0_mask)
            return _

        lax.fori_loop(0, n_ent, place_body, 0)

        def run_block(bl, col_base, slot, processed):
            st = offs[pl.ds(bl, 16)][0]
            ne = counts[pl.ds(bl, 16)][0]
            s16 = zeros + slot

            def e_body(e, pr):
                w = srt_w[pl.ds(e, 16)][0]
                pv = w & 16383
                r16 = zeros + (((w >> 14) & 127) + col_base)
                rslot = pr & (OUT_RING - 1)

                @pl.when(pr >= OUT_RING)
                def _drain_one():
                    pltpu.make_async_copy(
                        out_hbm.at[0], row_v.at[rslot], sem_out
                    ).wait()

                for k in range(DIM // 16):
                    c16 = lanes + (k * 16)
                    val = plsc.load_gather(chunk_v, [s16, c16, r16])
                    row_v[rslot, pl.ds(k * 16, 16)] = val
                pltpu.async_copy(row_v.at[rslot], out_hbm.at[pv], sem_out)
                return pr + 1

            return lax.fori_loop(st, st + ne, e_body, processed)

        n_ch = NB // 4 + last.astype(jnp.int32)

        def fire_chunk(ch):
            lane0 = pl.multiple_of((lo_b + ch * 4) << 7, 128)
            pltpu.async_copy(
                tab_t_hbm.at[:, pl.ds(lane0, CHW)],
                chunk_v.at[ch & (RING - 1)],
                sem,
            )

        def prologue(ch, _):
            fire_chunk(ch)
            return _

        lax.fori_loop(0, RING, prologue, 0)

        def chunk_body(ch, processed):
            slot = ch & (RING - 1)
            pltpu.make_async_copy(
                tab_t_hbm.at[:, pl.ds(0, CHW)], chunk_v.at[slot], sem
            ).wait()
            for sub in range(4):
                processed = run_block(
                    ch * 4 + sub, sub * 128, slot, processed
                )

            @pl.when(ch + RING < n_ch)
            def _refill():
                fire_chunk(ch + RING)

            return processed

        processed = lax.fori_loop(0, n_ch, chunk_body, 0)

        @pl.when(last)
        def _tail_block():
            lane_t = pl.multiple_of((NBLK - 1) * 128 + wid * 0, 128)
            pltpu.sync_copy(
                tab_t_hbm.at[:, pl.ds(lane_t, 128)],
                chunk_v.at[0, :, pl.ds(0, 128)],
            )

        processed = lax.cond(
            last,
            lambda p: run_block(NB + 4, 0, 0, p),
            lambda p: p,
            processed,
        )

        def drain_body(k, _):
            @pl.when(k < jnp.minimum(processed, OUT_RING))
            def _d():
                pltpu.make_async_copy(
                    out_hbm.at[0], row_v.at[k & (OUT_RING - 1)], sem_out
                ).wait()

            return _

        lax.fori_loop(0, OUT_RING, drain_body, 0)

    return gather_kernel


def kernel(problem_id, embedding_table):
    gather_kernel = _build()
    return gather_kernel(problem_id, embedding_table.T)


# speedup vs baseline: 2.6038x; 1.0157x over previous
import functools

import jax
import jax.numpy as jnp
from jax import lax
from jax.experimental import pallas as pl
from jax.experimental.pallas import tpu as pltpu
from jax.experimental.pallas import tpu_sc as plsc

BATCH = 16384
DIM = 64
NBLK = 7813
NB = 244
CHW = 512
RING = 2
OUT_RING = 16


@functools.cache
def _build():
    info = plsc.get_sparse_core_info()
    nc, ns = info.num_cores, info.num_subcores
    nw = nc * ns
    mesh = plsc.VectorSubcoreMesh(core_axis_name="c", subcore_axis_name="s")

    @functools.partial(
        pl.kernel,
        mesh=mesh,
        out_type=jax.ShapeDtypeStruct((BATCH, DIM), jnp.float32),
        scratch_types=[
            pltpu.VMEM((BATCH,), jnp.int32),
            pltpu.VMEM((BATCH + 16,), jnp.int32),
            pltpu.VMEM((BATCH + 16,), jnp.int32),
            pltpu.VMEM((80,), jnp.int32),
            pltpu.VMEM((80,), jnp.int32),
            pltpu.VMEM((80,), jnp.int32),
            pltpu.VMEM((RING, DIM, CHW), jnp.float32),
            pltpu.VMEM((OUT_RING, DIM), jnp.float32),
            pltpu.SemaphoreType.DMA,
            pltpu.SemaphoreType.DMA,
        ],
        compiler_params=pltpu.CompilerParams(needs_layout_passes=False),
    )
    def gather_kernel(idx_hbm, tab_t_hbm, out_hbm, idx_full, ent_w, srt_w,
                      counts, offs, fill, chunk_v, row_v, sem, sem_out):
        wid = lax.axis_index("s") * nc + lax.axis_index("c")
        last = wid == nw - 1
        lo_b = wid * NB
        n_b = NB + 5 * last.astype(jnp.int32)
        n_ch = NB // 4 + last.astype(jnp.int32)
        lanes = lax.iota(jnp.int32, 16)
        zeros = jnp.zeros((16,), jnp.int32)
        lane0_mask = lanes == 0

        def fire_chunk(ch):
            lane0 = pl.multiple_of((lo_b + ch * 4) << 7, 128)
            pltpu.async_copy(
                tab_t_hbm.at[:, pl.ds(lane0, CHW)],
                chunk_v.at[ch & (RING - 1)],
                sem,
            )

        def prologue(ch, _):
            fire_chunk(ch)
            return _

        lax.fori_loop(0, RING, prologue, 0)
        pltpu.sync_copy(idx_hbm, idx_full)

        def scan_body(g, n):
            ids = idx_full[pl.ds(g * 16, 16)]
            blk = ids >> 7
            m = (blk >= lo_b) & (blk < lo_b + n_b)
            rel = blk - lo_b
            packed = (
                ((rel >> 2) << 23)
                | ((((rel & 3) << 7) | (ids & 127)) << 14)
                | (g * 16 + lanes)
            )
            plsc.store_compressed(ent_w.at[pl.ds(n, 16)], packed, mask=m)
            return n + plsc.all_reduce_population_count(m)[0]

        n_ent = lax.fori_loop(0, BATCH // 16, scan_body, 0)

        for k in range(5):
            counts[pl.ds(k * 16, 16)] = zeros

        def count_body(e, _):
            b16 = zeros + (ent_w[pl.ds(e, 16)][0] >> 23)
            c = plsc.load_gather(counts, [b16])
            plsc.store_scatter(counts, [b16], c + 1, mask=lane0_mask)
            return _

        lax.fori_loop(0, n_ent, count_body, 0)

        carry = jnp.int32(0)
        for k in range(4):
            cv = counts[pl.ds(k * 16, 16)]
            s = plsc.cumsum(cv)
            start = s - cv + carry
            offs[pl.ds(k * 16, 16)] = start
            fill[pl.ds(k * 16, 16)] = start
            carry = carry + s[15]

        def place_body(e, _):
            w = ent_w[pl.ds(e, 16)][0]
            b16 = zeros + (w >> 23)
            o = plsc.load_gather(fill, [b16])
            plsc.store_scatter(srt_w, [o], zeros + w, mask=lane0_mask)
            plsc.store_scatter(fill, [b16], o + 1, mask=lane0_mask)
            return _

        lax.fori_loop(0, n_ent, place_body, 0)

        def run_chunk(ch, slot, processed):
            st = offs[pl.ds(ch, 16)][0]
            ne = counts[pl.ds(ch, 16)][0]
            s16 = zeros + slot

            def e_body(e, pr):
                w = srt_w[pl.ds(e, 16)][0]
                pv = w & 16383
                r16 = zeros + ((w >> 14) & 511)
                rslot = pr & (OUT_RING - 1)

                @pl.when(pr >= OUT_RING)
                def _drain_one():
                    pltpu.make_async_copy(
                        out_hbm.at[0], row_v.at[rslot], sem_out
                    ).wait()

                for k in range(DIM // 16):
                    c16 = lanes + (k * 16)
                    val = plsc.load_gather(chunk_v, [s16, c16, r16])
                    row_v[rslot, pl.ds(k * 16, 16)] = val
                pltpu.async_copy(row_v.at[rslot], out_hbm.at[pv], sem_out)
                return pr + 1

            return lax.fori_loop(st, st + ne, e_body, processed)

        def chunk_body(ch, processed):
            slot = ch & (RING - 1)
            pltpu.make_async_copy(
                tab_t_hbm.at[:, pl.ds(0, CHW)], chunk_v.at[slot], sem
            ).wait()
            processed = run_chunk(ch, slot, processed)

            @pl.when(ch + RING < n_ch)
            def _refill():
                fire_chunk(ch + RING)

            return processed

        processed = lax.fori_loop(0, n_ch, chunk_body, 0)

        @pl.when(last)
        def _tail_block():
            lane_t = pl.multiple_of((NBLK - 1) * 128 + wid * 0, 128)
            pltpu.sync_copy(
                tab_t_hbm.at[:, pl.ds(lane_t, 128)],
                chunk_v.at[0, :, pl.ds(0, 128)],
            )

        processed = lax.cond(
            last,
            lambda p: run_chunk(NB // 4 + 1, 0, p),
            lambda p: p,
            processed,
        )

        def drain_body(k, _):
            @pl.when(k < jnp.minimum(processed, OUT_RING))
            def _d():
                pltpu.make_async_copy(
                    out_hbm.at[0], row_v.at[k & (OUT_RING - 1)], sem_out
                ).wait()

            return _

        lax.fori_loop(0, OUT_RING, drain_body, 0)

    return gather_kernel


def kernel(problem_id, embedding_table):
    gather_kernel = _build()
    return gather_kernel(problem_id, embedding_table.T)
